# baseline (speedup 1.0000x reference)
import functools
import jax
import jax.numpy as jnp
from jax import lax
from jax.experimental import pallas as pl
from jax.experimental.pallas import tpu as pltpu
from jax.experimental.pallas import tpu_sc as plsc

N = 10000
E = 320000
D_FEAT = 128
D_EDGE = 16
E_HID = 64
OUT_CH = 128
INTER = 64

NC = 2
NS = 16
NW = NC * NS
EPW = E // NW
K = 40
NCH = EPW // K
N_PAD = 10112
RPT = N_PAD // NS


def _edge_mlp(edge_attr, We1, be1, We2, be2):
    BE = 8000

    def body(ea_ref, w1_ref, b1_ref, w2_ref, b2_ref, out_ref):
        t = jnp.dot(ea_ref[...], w1_ref[...], preferred_element_type=jnp.float32)
        t = jnp.maximum(t + b1_ref[...], 0.0)
        out_ref[...] = (
            jnp.dot(t, w2_ref[...], preferred_element_type=jnp.float32) + b2_ref[...]
        )

    return pl.pallas_call(
        body,
        grid=(E // BE,),
        in_specs=[
            pl.BlockSpec((BE, D_EDGE), lambda i: (i, 0)),
            pl.BlockSpec((D_EDGE, E_HID), lambda i: (0, 0)),
            pl.BlockSpec((1, E_HID), lambda i: (0, 0)),
            pl.BlockSpec((E_HID, D_FEAT), lambda i: (0, 0)),
            pl.BlockSpec((1, D_FEAT), lambda i: (0, 0)),
        ],
        out_specs=pl.BlockSpec((BE, D_FEAT), lambda i: (i, 0)),
        out_shape=jax.ShapeDtypeStruct((E, D_FEAT), jnp.float32),
    )(edge_attr, We1, be1.reshape(1, -1), We2, be2.reshape(1, -1))


def _sc_message(h, e, src, dst):
    mesh = plsc.VectorSubcoreMesh(core_axis_name="c", subcore_axis_name="s")

    def body(h_hbm, e_hbm, src_hbm, dst_hbm, out_hbm,
             si0, si1, si2, si3, di0, di1, di2, di3,
             hbuf0, ebuf0, hbuf1, ebuf1, agg_sh,
             isem0, isem1, isem2, isem3, dsem0, dsem1):
        cid = lax.axis_index("c")
        sid = lax.axis_index("s")
        wid = cid * NS + sid
        ebase = wid * EPW

        sis = (si0, si1, si2, si3)
        dis = (di0, di1, di2, di3)
        isems = (isem0, isem1, isem2, isem3)
        hbufs = (hbuf0, hbuf1)
        ebufs = (ebuf0, ebuf1)
        dsems = (dsem0, dsem1)

        def zrow(i, _):
            for c in range(D_FEAT // 16):
                ebuf0[i, pl.ds(c * 16, 16)] = jnp.zeros((16,), jnp.float32)
            return 0
        lax.fori_loop(0, K, zrow, 0, unroll=False)
        rb = sid * RPT
        for r in range(RPT // K):
            pltpu.sync_copy(ebuf0, agg_sh.at[pl.ds(rb + r * K, K)])
        pltpu.sync_copy(ebuf0.at[pl.ds(0, RPT - (RPT // K) * K)],
                        agg_sh.at[pl.ds(rb + (RPT // K) * K,
                                        RPT - (RPT // K) * K)])
        plsc.subcore_barrier()

        def start_idx(j, isl):
            base = ebase + j * K
            pltpu.async_copy(src_hbm.at[pl.ds(base, K)], sis[isl], isems[isl])
            pltpu.async_copy(dst_hbm.at[pl.ds(base, K)], dis[isl], isems[isl])

        def start_data(j, isl, dsl):
            base = ebase + j * K
            pltpu.make_async_copy(src_hbm.at[pl.ds(base, K)], sis[isl],
                                  isems[isl]).wait()
            pltpu.make_async_copy(dst_hbm.at[pl.ds(base, K)], dis[isl],
                                  isems[isl]).wait()
            pltpu.async_copy(h_hbm.at[sis[isl]], hbufs[dsl], dsems[dsl])
            pltpu.async_copy(e_hbm.at[pl.ds(base, K)], ebufs[dsl], dsems[dsl])

        def finish(j, isl, dsl):
            hb, eb = hbufs[dsl], ebufs[dsl]
            base = ebase + j * K
            pltpu.make_async_copy(h_hbm.at[pl.ds(0, K)], hb, dsems[dsl]).wait()
            pltpu.make_async_copy(e_hbm.at[pl.ds(base, K)], eb,
                                  dsems[dsl]).wait()

            def row(i, _):
                for c in range(D_FEAT // 16):
                    sl = pl.ds(c * 16, 16)
                    eb[i, sl] = jnp.maximum(hb[i, sl] + eb[i, sl], 0.0)
                return 0
            lax.fori_loop(0, K, row, 0, unroll=5)

            pltpu.sync_copy(eb, agg_sh.at[dis[isl]], add=True)

        def step(j, k):
            finish(j, k, k % 2)

            @pl.when(j + 4 < NCH)
            def _():
                start_idx(j + 4, k)

            @pl.when(j + 2 < NCH)
            def _():
                start_data(j + 2, (k + 2) % 4, k % 2)

        for k in range(4):
            start_idx(k, k)
        start_data(0, 0, 0)
        start_data(1, 1, 1)

        def quad(t, _):
            j = 4 * t
            for k in range(4):
                step(j + k, k)
            return 0
        lax.fori_loop(0, NCH // 4, quad, 0, unroll=False)
        step(NCH - 2, 0)
        step(NCH - 1, 1)

        plsc.subcore_barrier()
        pltpu.sync_copy(agg_sh.at[pl.ds(sid * RPT, RPT)],
                        out_hbm.at[cid, pl.ds(sid * RPT, RPT)])

    f = pl.kernel(
        body,
        out_type=jax.ShapeDtypeStruct((NC, N_PAD, D_FEAT), jnp.float32),
        mesh=mesh,
        scratch_types=(
            [pltpu.VMEM((K,), jnp.int32)] * 8
            + [pltpu.VMEM((K, D_FEAT), jnp.float32)] * 4
            + [pltpu.VMEM_SHARED((N_PAD, D_FEAT), jnp.float32)]
            + [pltpu.SemaphoreType.DMA] * 6
        ),
    )
    return f(h, e, src, dst)


def _node_mlp(agg2, h, Wn1, bn1, Wn2, bn2):
    BN = 2000

    def body(agg_ref, h_ref, w1_ref, b1_ref, w2_ref, b2_ref, out_ref):
        out = agg_ref[0] + agg_ref[1] + h_ref[...]
        t = jnp.dot(out, w1_ref[...], preferred_element_type=jnp.float32)
        t = jnp.maximum(t + b1_ref[...], 0.0)
        out_ref[...] = (
            jnp.dot(t, w2_ref[...], preferred_element_type=jnp.float32) + b2_ref[...]
        )

    return pl.pallas_call(
        body,
        grid=(N // BN,),
        in_specs=[
            pl.BlockSpec((NC, BN, D_FEAT), lambda i: (0, i, 0)),
            pl.BlockSpec((BN, D_FEAT), lambda i: (i, 0)),
            pl.BlockSpec((D_FEAT, OUT_CH), lambda i: (0, 0)),
            pl.BlockSpec((1, OUT_CH), lambda i: (0, 0)),
            pl.BlockSpec((OUT_CH, OUT_CH), lambda i: (0, 0)),
            pl.BlockSpec((1, OUT_CH), lambda i: (0, 0)),
        ],
        out_specs=pl.BlockSpec((BN, OUT_CH), lambda i: (i, 0)),
        out_shape=jax.ShapeDtypeStruct((N, OUT_CH), jnp.float32),
    )(agg2, h, Wn1, bn1.reshape(1, -1), Wn2, bn2.reshape(1, -1))


def _node_mlp_final(agg2, h, Wn1, bn1, Wn2, bn2, Wagg):
    BN = 2000
    G = N // BN

    def body(agg_ref, h_ref, w1_ref, b1_ref, w2_ref, b2_ref, wagg_ref, out_ref,
             acc_ref):
        i = pl.program_id(0)
        out = agg_ref[0] + agg_ref[1] + h_ref[...]
        t = jnp.dot(out, w1_ref[...], preferred_element_type=jnp.float32)
        t = jnp.maximum(t + b1_ref[...], 0.0)
        hn = jnp.dot(t, w2_ref[...], preferred_element_type=jnp.float32) + b2_ref[...]

        @pl.when(i == 0)
        def _():
            acc_ref[...] = jnp.zeros_like(acc_ref)

        acc_ref[...] += jnp.sum(hn, axis=0, keepdims=True)

        @pl.when(i == G - 1)
        def _():
            out_ref[...] = jnp.maximum(
                jnp.dot(acc_ref[...], wagg_ref[...],
                        preferred_element_type=jnp.float32), 0.0)

    return pl.pallas_call(
        body,
        grid=(G,),
        in_specs=[
            pl.BlockSpec((NC, BN, D_FEAT), lambda i: (0, i, 0)),
            pl.BlockSpec((BN, D_FEAT), lambda i: (i, 0)),
            pl.BlockSpec((D_FEAT, OUT_CH), lambda i: (0, 0)),
            pl.BlockSpec((1, OUT_CH), lambda i: (0, 0)),
            pl.BlockSpec((OUT_CH, OUT_CH), lambda i: (0, 0)),
            pl.BlockSpec((1, OUT_CH), lambda i: (0, 0)),
            pl.BlockSpec((OUT_CH, INTER), lambda i: (0, 0)),
        ],
        out_specs=pl.BlockSpec((1, INTER), lambda i: (0, 0)),
        out_shape=jax.ShapeDtypeStruct((1, INTER), jnp.float32),
        scratch_shapes=[pltpu.VMEM((1, OUT_CH), jnp.float32)],
    )(agg2, h, Wn1, bn1.reshape(1, -1), Wn2, bn2.reshape(1, -1), Wagg)


def kernel(x, edge_index, edge_attr, params):
    src = edge_index[0]
    dst = edge_index[1]
    h = x
    layers = params["layers"]
    for l, p in enumerate(layers):
        e = _edge_mlp(edge_attr, p["We1"], p["be1"], p["We2"], p["be2"])
        agg2 = _sc_message(h, e, src, dst)
        if l == len(layers) - 1:
            return _node_mlp_final(agg2, h, p["Wn1"], p["bn1"], p["Wn2"], p["bn2"],
                                   params["Wagg"])
        h = _node_mlp(agg2, h, p["Wn1"], p["bn1"], p["Wn2"], p["bn2"])


# speedup vs baseline: 1.6538x; 1.6538x over previous
import functools
import jax
import jax.numpy as jnp
from jax import lax
from jax.experimental import pallas as pl
from jax.experimental.pallas import tpu as pltpu
from jax.experimental.pallas import tpu_sc as plsc

N = 10000
E = 320000
D_FEAT = 128
D_EDGE = 16
E_HID = 64
OUT_CH = 128
INTER = 64

NC = 2
NS = 16
NW = NC * NS
EPW = E // NW
K = 40
NCH = EPW // K
N_PAD = 10112
RPT = N_PAD // NS


def _edge_mlp(edge_attr, We1, be1, We2, be2):
    BE = 8000

    def body(ea_ref, w1_ref, b1_ref, w2_ref, b2_ref, out_ref):
        t = jnp.dot(ea_ref[...], w1_ref[...], preferred_element_type=jnp.float32)
        t = jnp.maximum(t + b1_ref[...], 0.0)
        out_ref[...] = (
            jnp.dot(t, w2_ref[...], preferred_element_type=jnp.float32) + b2_ref[...]
        )

    return pl.pallas_call(
        body,
        grid=(E // BE,),
        in_specs=[
            pl.BlockSpec((BE, D_EDGE), lambda i: (i, 0)),
            pl.BlockSpec((D_EDGE, E_HID), lambda i: (0, 0)),
            pl.BlockSpec((1, E_HID), lambda i: (0, 0)),
            pl.BlockSpec((E_HID, D_FEAT), lambda i: (0, 0)),
            pl.BlockSpec((1, D_FEAT), lambda i: (0, 0)),
        ],
        out_specs=pl.BlockSpec((BE, D_FEAT), lambda i: (i, 0)),
        out_shape=jax.ShapeDtypeStruct((E, D_FEAT), jnp.float32),
    )(edge_attr, We1, be1.reshape(1, -1), We2, be2.reshape(1, -1))


def _sc_message(h, e, src, dst):
    mesh = plsc.VectorSubcoreMesh(core_axis_name="c", subcore_axis_name="s")

    def body(h_hbm, e_hbm, src_hbm, dst_hbm, out_hbm,
             si0, si1, si2, si3, di0, di1, di2, di3,
             hbuf0, ebuf0, hbuf1, ebuf1, agg_sh,
             isem0, isem1, isem2, isem3, dsem0, dsem1):
        cid = lax.axis_index("c")
        sid = lax.axis_index("s")
        wid = cid * NS + sid
        ebase = wid * EPW

        sis = (si0, si1, si2, si3)
        dis = (di0, di1, di2, di3)
        isems = (isem0, isem1, isem2, isem3)
        hbufs = (hbuf0, hbuf1)
        ebufs = (ebuf0, ebuf1)
        dsems = (dsem0, dsem1)

        def zrow(i, _):
            for c in range(D_FEAT // 16):
                ebuf0[i, pl.ds(c * 16, 16)] = jnp.zeros((16,), jnp.float32)
            return 0
        lax.fori_loop(0, K, zrow, 0, unroll=False)
        rb = sid * RPT
        for r in range(RPT // K):
            pltpu.sync_copy(ebuf0, agg_sh.at[pl.ds(rb + r * K, K)])
        pltpu.sync_copy(ebuf0.at[pl.ds(0, RPT - (RPT // K) * K)],
                        agg_sh.at[pl.ds(rb + (RPT // K) * K,
                                        RPT - (RPT // K) * K)])
        plsc.subcore_barrier()

        def start_idx(j, isl):
            base = ebase + j * K
            pltpu.async_copy(src_hbm.at[pl.ds(base, K)], sis[isl], isems[isl])
            pltpu.async_copy(dst_hbm.at[pl.ds(base, K)], dis[isl], isems[isl])

        def start_data(j, isl, dsl):
            base = ebase + j * K
            pltpu.make_async_copy(src_hbm.at[pl.ds(base, K)], sis[isl],
                                  isems[isl]).wait()
            pltpu.make_async_copy(dst_hbm.at[pl.ds(base, K)], dis[isl],
                                  isems[isl]).wait()
            pltpu.async_copy(h_hbm.at[sis[isl]], hbufs[dsl], dsems[dsl])
            pltpu.async_copy(e_hbm.at[pl.ds(base, K)], ebufs[dsl], dsems[dsl])

        def finish(j, isl, dsl):
            hb, eb = hbufs[dsl], ebufs[dsl]
            base = ebase + j * K
            pltpu.make_async_copy(h_hbm.at[pl.ds(0, K)], hb, dsems[dsl]).wait()
            pltpu.make_async_copy(e_hbm.at[pl.ds(base, K)], eb,
                                  dsems[dsl]).wait()

            def row(i, _):
                for c in range(D_FEAT // 16):
                    sl = pl.ds(c * 16, 16)
                    eb[i, sl] = jnp.maximum(hb[i, sl] + eb[i, sl], 0.0)
                return 0
            lax.fori_loop(0, K, row, 0, unroll=False)

            pltpu.sync_copy(eb, agg_sh.at[dis[isl]], add=True)

        def step(j, k):
            finish(j, k, k % 2)

            @pl.when(j + 4 < NCH)
            def _():
                start_idx(j + 4, k)

            @pl.when(j + 2 < NCH)
            def _():
                start_data(j + 2, (k + 2) % 4, k % 2)

        for k in range(4):
            start_idx(k, k)
        start_data(0, 0, 0)
        start_data(1, 1, 1)

        def quad(t, _):
            j = 4 * t
            for k in range(4):
                step(j + k, k)
            return 0
        lax.fori_loop(0, NCH // 4, quad, 0, unroll=False)
        step(NCH - 2, 0)
        step(NCH - 1, 1)

        plsc.subcore_barrier()
        pltpu.sync_copy(agg_sh.at[pl.ds(sid * RPT, RPT)],
                        out_hbm.at[cid, pl.ds(sid * RPT, RPT)])

    f = pl.kernel(
        body,
        out_type=jax.ShapeDtypeStruct((NC, N_PAD, D_FEAT), jnp.float32),
        mesh=mesh,
        scratch_types=(
            [pltpu.VMEM((K,), jnp.int32)] * 8
            + [pltpu.VMEM((K, D_FEAT), jnp.float32)] * 4
            + [pltpu.VMEM_SHARED((N_PAD, D_FEAT), jnp.float32)]
            + [pltpu.SemaphoreType.DMA] * 6
        ),
    )
    return f(h, e, src, dst)


def _node_mlp(agg2, h, Wn1, bn1, Wn2, bn2):
    BN = 2000

    def body(agg_ref, h_ref, w1_ref, b1_ref, w2_ref, b2_ref, out_ref):
        out = agg_ref[0] + agg_ref[1] + h_ref[...]
        t = jnp.dot(out, w1_ref[...], preferred_element_type=jnp.float32)
        t = jnp.maximum(t + b1_ref[...], 0.0)
        out_ref[...] = (
            jnp.dot(t, w2_ref[...], preferred_element_type=jnp.float32) + b2_ref[...]
        )

    return pl.pallas_call(
        body,
        grid=(N // BN,),
        in_specs=[
            pl.BlockSpec((NC, BN, D_FEAT), lambda i: (0, i, 0)),
            pl.BlockSpec((BN, D_FEAT), lambda i: (i, 0)),
            pl.BlockSpec((D_FEAT, OUT_CH), lambda i: (0, 0)),
            pl.BlockSpec((1, OUT_CH), lambda i: (0, 0)),
            pl.BlockSpec((OUT_CH, OUT_CH), lambda i: (0, 0)),
            pl.BlockSpec((1, OUT_CH), lambda i: (0, 0)),
        ],
        out_specs=pl.BlockSpec((BN, OUT_CH), lambda i: (i, 0)),
        out_shape=jax.ShapeDtypeStruct((N, OUT_CH), jnp.float32),
    )(agg2, h, Wn1, bn1.reshape(1, -1), Wn2, bn2.reshape(1, -1))


def _node_mlp_final(agg2, h, Wn1, bn1, Wn2, bn2, Wagg):
    BN = 2000
    G = N // BN

    def body(agg_ref, h_ref, w1_ref, b1_ref, w2_ref, b2_ref, wagg_ref, out_ref,
             acc_ref):
        i = pl.program_id(0)
        out = agg_ref[0] + agg_ref[1] + h_ref[...]
        t = jnp.dot(out, w1_ref[...], preferred_element_type=jnp.float32)
        t = jnp.maximum(t + b1_ref[...], 0.0)
        hn = jnp.dot(t, w2_ref[...], preferred_element_type=jnp.float32) + b2_ref[...]

        @pl.when(i == 0)
        def _():
            acc_ref[...] = jnp.zeros_like(acc_ref)

        acc_ref[...] += jnp.sum(hn, axis=0, keepdims=True)

        @pl.when(i == G - 1)
        def _():
            out_ref[...] = jnp.maximum(
                jnp.dot(acc_ref[...], wagg_ref[...],
                        preferred_element_type=jnp.float32), 0.0)

    return pl.pallas_call(
        body,
        grid=(G,),
        in_specs=[
            pl.BlockSpec((NC, BN, D_FEAT), lambda i: (0, i, 0)),
            pl.BlockSpec((BN, D_FEAT), lambda i: (i, 0)),
            pl.BlockSpec((D_FEAT, OUT_CH), lambda i: (0, 0)),
            pl.BlockSpec((1, OUT_CH), lambda i: (0, 0)),
            pl.BlockSpec((OUT_CH, OUT_CH), lambda i: (0, 0)),
            pl.BlockSpec((1, OUT_CH), lambda i: (0, 0)),
            pl.BlockSpec((OUT_CH, INTER), lambda i: (0, 0)),
        ],
        out_specs=pl.BlockSpec((1, INTER), lambda i: (0, 0)),
        out_shape=jax.ShapeDtypeStruct((1, INTER), jnp.float32),
        scratch_shapes=[pltpu.VMEM((1, OUT_CH), jnp.float32)],
    )(agg2, h, Wn1, bn1.reshape(1, -1), Wn2, bn2.reshape(1, -1), Wagg)


def kernel(x, edge_index, edge_attr, params):
    src = edge_index[0]
    dst = edge_index[1]
    h = x
    layers = params["layers"]
    es = [_edge_mlp(edge_attr, p["We1"], p["be1"], p["We2"], p["be2"])
          for p in layers]
    for l, p in enumerate(layers):
        agg2 = _sc_message(h, es[l], src, dst)
        if l == len(layers) - 1:
            return _node_mlp_final(agg2, h, p["Wn1"], p["bn1"], p["Wn2"], p["bn2"],
                                   params["Wagg"])
        h = _node_mlp(agg2, h, p["Wn1"], p["bn1"], p["Wn2"], p["bn2"])
